# baseline (speedup 1.0000x reference)
import functools

import jax
import jax.numpy as jnp
from jax import lax
from jax.experimental import pallas as pl
from jax.experimental.pallas import tpu as pltpu

N_NEIGH = 40
K_SEL = N_NEIGH - 1
N_DIM = 4
N_PROP = 64
N_FILT = 64
IN_DIM = 64
B = 8
S = 1024
N = B * S

_INF_BITS = 0x7F800000
_NEG_BIG = -1e30


def _segment_kernel(x_ref, xt_ref, wsp_ref, wst_ref, wf_ref, bf_ref,
                    wo1_ref, wo2_ref, wo3_ref, bo_ref, out_ref,
                    feats_ref, wmt_ref):
    x = x_ref[...]
    xt = xt_ref[...]

    cpad = jnp.dot(x, wsp_ref[...], preferred_element_type=jnp.float32)
    cpadt = jnp.dot(wst_ref[...], xt, preferred_element_type=jnp.float32)
    g = jnp.dot(cpad, cpadt, preferred_element_type=jnp.float32)

    row_i = lax.broadcasted_iota(jnp.int32, (S, S), 0)
    col_i = lax.broadcasted_iota(jnp.int32, (S, S), 1)
    eye = row_i == col_i
    gz = jnp.where(eye, g, 0.0)
    sq_col = jnp.sum(gz, axis=1, keepdims=True)
    sq_row = jnp.sum(gz, axis=0, keepdims=True)

    d2 = jnp.maximum(sq_col + sq_row - 2.0 * g, 0.0)
    d2 = jnp.where(eye, jnp.inf, d2)
    dbits = lax.bitcast_convert_type(d2, jnp.int32)

    def bs_body(_, carry):
        lo, hi = carry
        mid = lo + ((hi - lo) >> 1)
        cnt = jnp.sum((dbits <= mid).astype(jnp.int32), axis=1, keepdims=True)
        ge = cnt >= K_SEL
        return jnp.where(ge, lo, mid), jnp.where(ge, mid, hi)

    lo0 = jnp.full((S, 1), -1, jnp.int32)
    hi0 = jnp.full((S, 1), _INF_BITS, jnp.int32)
    _, t_col = lax.fori_loop(0, 31, bs_body, (lo0, hi0))

    t_row = jnp.sum(jnp.where(eye, jnp.broadcast_to(t_col, (S, S)), 0),
                    axis=0, keepdims=True)

    w = jnp.exp(-(d2 * 10.0 + 1e-5))
    sel_row = dbits <= t_row
    wmt_ref[...] = jnp.where(sel_row, w, 0.0)

    feats = jnp.dot(x, wf_ref[...], preferred_element_type=jnp.float32) + bf_ref[...]
    feats_ref[...] = feats

    mean_p = lax.dot_general(wmt_ref[...], feats,
                             (((0,), (0,)), ((), ())),
                             preferred_element_type=jnp.float32) * (1.0 / K_SEL)

    CH = 512

    def make_body(c4):
        def max_body(jb, acc):
            f8 = feats_ref[pl.ds(jb * 8, 8), :]
            f8t = f8.T
            w8 = wmt_ref[pl.ds(jb * 8, 8), pl.ds(c4 * CH, CH)]
            for k in range(8):
                fk = f8t[:, k:k + 1]
                wr = w8[k:k + 1, :]
                ar = jnp.where(wr > 0.0, 0.0, _NEG_BIG)
                acc = jnp.maximum(acc, fk * wr + ar)
            return acc
        return max_body

    parts = []
    for c4 in range(S // CH):
        acc0 = jnp.full((N_PROP, CH), _NEG_BIG, jnp.float32)
        parts.append(lax.fori_loop(0, S // 8, make_body(c4), acc0))
    acct = jnp.concatenate(parts, axis=1)

    out = jnp.dot(x, wo1_ref[...], preferred_element_type=jnp.float32)
    out += lax.dot_general(acct, wo2_ref[...],
                           (((0,), (0,)), ((), ())),
                           preferred_element_type=jnp.float32)
    out += jnp.dot(mean_p, wo3_ref[...], preferred_element_type=jnp.float32)
    out_ref[...] = jnp.tanh(out + bo_ref[...])


@jax.jit
def kernel(x, row_splits, Ws, bs, Wf, bf, Wo, bo):
    del row_splits, bs
    wsp = jnp.zeros((IN_DIM, 128), jnp.float32).at[:, :N_DIM].set(Ws)
    wst = jnp.zeros((128, IN_DIM), jnp.float32).at[:N_DIM, :].set(Ws.T)
    xt = x.T
    wo1 = Wo[:IN_DIM]
    wo2 = Wo[IN_DIM:IN_DIM + N_PROP]
    wo3 = Wo[IN_DIM + N_PROP:]

    grid = (B,)
    full = lambda shape: pl.BlockSpec(shape, lambda i: (0, 0))
    out = pl.pallas_call(
        _segment_kernel,
        grid=grid,
        in_specs=[
            pl.BlockSpec((S, IN_DIM), lambda i: (i, 0)),
            pl.BlockSpec((IN_DIM, S), lambda i: (0, i)),
            full((IN_DIM, 128)),
            full((128, IN_DIM)),
            full((IN_DIM, N_PROP)),
            full((1, N_PROP)),
            full((IN_DIM, N_FILT)),
            full((N_PROP, N_FILT)),
            full((N_PROP, N_FILT)),
            full((1, N_FILT)),
        ],
        out_specs=pl.BlockSpec((S, N_FILT), lambda i: (i, 0)),
        out_shape=jax.ShapeDtypeStruct((N, N_FILT), jnp.float32),
        scratch_shapes=[
            pltpu.VMEM((S, N_PROP), jnp.float32),
            pltpu.VMEM((S, S), jnp.float32),
        ],
    )(x, xt, wsp, wst, Wf, bf.reshape(1, N_PROP), wo1, wo2, wo3,
      bo.reshape(1, N_FILT))
    return out


# speedup vs baseline: 1.0026x; 1.0026x over previous
import functools

import jax
import jax.numpy as jnp
from jax import lax
from jax.experimental import pallas as pl
from jax.experimental.pallas import tpu as pltpu

N_NEIGH = 40
K_SEL = N_NEIGH - 1
N_DIM = 4
N_PROP = 64
N_FILT = 64
IN_DIM = 64
B = 8
S = 1024
N = B * S

_INF_BITS = 0x7F800000
_NEG_BIG = -1e30


def _segment_kernel(x_ref, xt_ref, wsp_ref, wst_ref, wf_ref, bf_ref,
                    wo1_ref, wo2_ref, wo3_ref, bo_ref, out_ref,
                    feats_ref, wmt_ref, amt_ref):
    x = x_ref[...]
    xt = xt_ref[...]

    cpad = jnp.dot(x, wsp_ref[...], preferred_element_type=jnp.float32)
    cpadt = jnp.dot(wst_ref[...], xt, preferred_element_type=jnp.float32)
    g = jnp.dot(cpad, cpadt, preferred_element_type=jnp.float32)

    row_i = lax.broadcasted_iota(jnp.int32, (S, S), 0)
    col_i = lax.broadcasted_iota(jnp.int32, (S, S), 1)
    eye = row_i == col_i
    gz = jnp.where(eye, g, 0.0)
    sq_col = jnp.sum(gz, axis=1, keepdims=True)
    sq_row = jnp.sum(gz, axis=0, keepdims=True)

    d2 = jnp.maximum(sq_col + sq_row - 2.0 * g, 0.0)
    d2 = jnp.where(eye, jnp.inf, d2)
    dbits = lax.bitcast_convert_type(d2, jnp.int32)

    def bs_body(_, carry):
        lo, hi = carry
        mid = lo + ((hi - lo) >> 1)
        cnt = jnp.sum((dbits <= mid).astype(jnp.int32), axis=1, keepdims=True)
        ge = cnt >= K_SEL
        return jnp.where(ge, lo, mid), jnp.where(ge, mid, hi)

    lo0 = jnp.full((S, 1), -1, jnp.int32)
    hi0 = jnp.full((S, 1), _INF_BITS, jnp.int32)
    _, t_col = lax.fori_loop(0, 31, bs_body, (lo0, hi0))

    t_row = jnp.sum(jnp.where(eye, jnp.broadcast_to(t_col, (S, S)), 0),
                    axis=0, keepdims=True)

    w = jnp.exp(-(d2 * 10.0 + 1e-5))
    sel_row = dbits <= t_row
    wmt_ref[...] = jnp.where(sel_row, w, 0.0)
    amt_ref[...] = jnp.where(sel_row, 0.0, _NEG_BIG)

    feats = jnp.dot(x, wf_ref[...], preferred_element_type=jnp.float32) + bf_ref[...]
    feats_ref[...] = feats

    mean_p = lax.dot_general(wmt_ref[...], feats,
                             (((0,), (0,)), ((), ())),
                             preferred_element_type=jnp.float32) * (1.0 / K_SEL)

    CH = 512

    def make_body(c4):
        def max_body(jb, acc):
            f8 = feats_ref[pl.ds(jb * 8, 8), :]
            f8t = f8.T
            w8 = wmt_ref[pl.ds(jb * 8, 8), pl.ds(c4 * CH, CH)]
            a8 = amt_ref[pl.ds(jb * 8, 8), pl.ds(c4 * CH, CH)]
            for k in range(8):
                fk = f8t[:, k:k + 1]
                wr = w8[k:k + 1, :]
                ar = a8[k:k + 1, :]
                acc = jnp.maximum(acc, fk * wr + ar)
            return acc
        return max_body

    parts = []
    for c4 in range(S // CH):
        acc0 = jnp.full((N_PROP, CH), _NEG_BIG, jnp.float32)
        parts.append(lax.fori_loop(0, S // 8, make_body(c4), acc0))
    acct = jnp.concatenate(parts, axis=1)

    out = jnp.dot(x, wo1_ref[...], preferred_element_type=jnp.float32)
    out += lax.dot_general(acct, wo2_ref[...],
                           (((0,), (0,)), ((), ())),
                           preferred_element_type=jnp.float32)
    out += jnp.dot(mean_p, wo3_ref[...], preferred_element_type=jnp.float32)
    out_ref[...] = jnp.tanh(out + bo_ref[...])


@jax.jit
def kernel(x, row_splits, Ws, bs, Wf, bf, Wo, bo):
    del row_splits, bs
    wsp = jnp.zeros((IN_DIM, 128), jnp.float32).at[:, :N_DIM].set(Ws)
    wst = jnp.zeros((128, IN_DIM), jnp.float32).at[:N_DIM, :].set(Ws.T)
    xt = x.T
    wo1 = Wo[:IN_DIM]
    wo2 = Wo[IN_DIM:IN_DIM + N_PROP]
    wo3 = Wo[IN_DIM + N_PROP:]

    grid = (B,)
    full = lambda shape: pl.BlockSpec(shape, lambda i: (0, 0))
    out = pl.pallas_call(
        _segment_kernel,
        grid=grid,
        in_specs=[
            pl.BlockSpec((S, IN_DIM), lambda i: (i, 0)),
            pl.BlockSpec((IN_DIM, S), lambda i: (0, i)),
            full((IN_DIM, 128)),
            full((128, IN_DIM)),
            full((IN_DIM, N_PROP)),
            full((1, N_PROP)),
            full((IN_DIM, N_FILT)),
            full((N_PROP, N_FILT)),
            full((N_PROP, N_FILT)),
            full((1, N_FILT)),
        ],
        out_specs=pl.BlockSpec((S, N_FILT), lambda i: (i, 0)),
        out_shape=jax.ShapeDtypeStruct((N, N_FILT), jnp.float32),
        scratch_shapes=[
            pltpu.VMEM((S, N_PROP), jnp.float32),
            pltpu.VMEM((S, S), jnp.float32),
            pltpu.VMEM((S, S), jnp.float32),
        ],
    )(x, xt, wsp, wst, Wf, bf.reshape(1, N_PROP), wo1, wo2, wo3,
      bo.reshape(1, N_FILT))
    return out


# speedup vs baseline: 1.5428x; 1.5388x over previous
import functools

import jax
import jax.numpy as jnp
from jax import lax
from jax.experimental import pallas as pl
from jax.experimental.pallas import tpu as pltpu

N_NEIGH = 40
K_SEL = N_NEIGH - 1
N_DIM = 4
N_PROP = 64
N_FILT = 64
IN_DIM = 64
B = 8
S = 1024
N = B * S

_INF_BITS = 0x7F800000
_NEG_BIG = -1e30


def _segment_kernel(x_ref, xt_ref, wsp_ref, wst_ref, wf_ref, bf_ref,
                    wo1_ref, wo2_ref, wo3_ref, bo_ref, out_ref,
                    feats16_ref, wmt_ref, wmt16_ref, amt16_ref):
    x = x_ref[...]
    xt = xt_ref[...]

    cpad = jnp.dot(x, wsp_ref[...], preferred_element_type=jnp.float32)
    cpadt = jnp.dot(wst_ref[...], xt, preferred_element_type=jnp.float32)
    g = jnp.dot(cpad, cpadt, preferred_element_type=jnp.float32)

    row_i = lax.broadcasted_iota(jnp.int32, (S, S), 0)
    col_i = lax.broadcasted_iota(jnp.int32, (S, S), 1)
    eye = row_i == col_i
    gz = jnp.where(eye, g, 0.0)
    sq_col = jnp.sum(gz, axis=1, keepdims=True)
    sq_row = jnp.sum(gz, axis=0, keepdims=True)

    d2 = jnp.maximum(sq_col + sq_row - 2.0 * g, 0.0)
    d2 = jnp.where(eye, jnp.inf, d2)
    dbits = lax.bitcast_convert_type(d2, jnp.int32)

    def bs_body(_, carry):
        lo, hi = carry
        mid = lo + ((hi - lo) >> 1)
        cnt = jnp.sum((dbits <= mid).astype(jnp.int32), axis=1, keepdims=True)
        ge = cnt >= K_SEL
        return jnp.where(ge, lo, mid), jnp.where(ge, mid, hi)

    lo0 = jnp.full((S, 1), -1, jnp.int32)
    hi0 = jnp.full((S, 1), _INF_BITS, jnp.int32)
    _, t_col = lax.fori_loop(0, 31, bs_body, (lo0, hi0))

    t_row = jnp.sum(jnp.where(eye, jnp.broadcast_to(t_col, (S, S)), 0),
                    axis=0, keepdims=True)

    w = jnp.exp(-(d2 * 10.0 + 1e-5))
    sel_row = dbits <= t_row
    wmt_ref[...] = jnp.where(sel_row, w, 0.0)
    wmt16_ref[...] = jnp.where(sel_row, w, 0.0).astype(jnp.bfloat16)
    amt16_ref[...] = jnp.where(sel_row, 0.0, _NEG_BIG).astype(jnp.bfloat16)

    feats = jnp.dot(x, wf_ref[...], preferred_element_type=jnp.float32) + bf_ref[...]
    feats16_ref[...] = feats.astype(jnp.bfloat16)

    mean_p = lax.dot_general(wmt_ref[...], feats,
                             (((0,), (0,)), ((), ())),
                             preferred_element_type=jnp.float32) * (1.0 / K_SEL)

    CH = 512

    def make_body(c4):
        def max_body(jb, acc):
            f8 = feats16_ref[pl.ds(jb * 16, 16), :]
            f8t = f8.T
            w8 = wmt16_ref[pl.ds(jb * 16, 16), pl.ds(c4 * CH, CH)]
            a8 = amt16_ref[pl.ds(jb * 16, 16), pl.ds(c4 * CH, CH)]
            for k in range(16):
                fk = f8t[:, k:k + 1]
                wr = w8[k:k + 1, :]
                ar = a8[k:k + 1, :]
                acc = jnp.maximum(acc, fk * wr + ar)
            return acc
        return max_body

    parts = []
    for c4 in range(S // CH):
        acc0 = jnp.full((N_PROP, CH), _NEG_BIG, jnp.bfloat16)
        parts.append(lax.fori_loop(0, S // 16, make_body(c4), acc0))
    acct = jnp.concatenate(parts, axis=1).astype(jnp.float32)

    out = jnp.dot(x, wo1_ref[...], preferred_element_type=jnp.float32)
    out += lax.dot_general(acct, wo2_ref[...],
                           (((0,), (0,)), ((), ())),
                           preferred_element_type=jnp.float32)
    out += jnp.dot(mean_p, wo3_ref[...], preferred_element_type=jnp.float32)
    out_ref[...] = jnp.tanh(out + bo_ref[...])


@jax.jit
def kernel(x, row_splits, Ws, bs, Wf, bf, Wo, bo):
    del row_splits, bs
    wsp = jnp.zeros((IN_DIM, 128), jnp.float32).at[:, :N_DIM].set(Ws)
    wst = jnp.zeros((128, IN_DIM), jnp.float32).at[:N_DIM, :].set(Ws.T)
    xt = x.T
    wo1 = Wo[:IN_DIM]
    wo2 = Wo[IN_DIM:IN_DIM + N_PROP]
    wo3 = Wo[IN_DIM + N_PROP:]

    grid = (B,)
    full = lambda shape: pl.BlockSpec(shape, lambda i: (0, 0))
    out = pl.pallas_call(
        _segment_kernel,
        grid=grid,
        in_specs=[
            pl.BlockSpec((S, IN_DIM), lambda i: (i, 0)),
            pl.BlockSpec((IN_DIM, S), lambda i: (0, i)),
            full((IN_DIM, 128)),
            full((128, IN_DIM)),
            full((IN_DIM, N_PROP)),
            full((1, N_PROP)),
            full((IN_DIM, N_FILT)),
            full((N_PROP, N_FILT)),
            full((N_PROP, N_FILT)),
            full((1, N_FILT)),
        ],
        out_specs=pl.BlockSpec((S, N_FILT), lambda i: (i, 0)),
        out_shape=jax.ShapeDtypeStruct((N, N_FILT), jnp.float32),
        scratch_shapes=[
            pltpu.VMEM((S, N_PROP), jnp.bfloat16),
            pltpu.VMEM((S, S), jnp.float32),
            pltpu.VMEM((S, S), jnp.bfloat16),
            pltpu.VMEM((S, S), jnp.bfloat16),
        ],
    )(x, xt, wsp, wst, Wf, bf.reshape(1, N_PROP), wo1, wo2, wo3,
      bo.reshape(1, N_FILT))
    return out
